# Initial kernel scaffold; baseline (speedup 1.0000x reference)
import jax
import jax.numpy as jnp
from jax.experimental import pallas as pl


def kernel(edge_index, edge_weight, feats):
    raise NotImplementedError("write your pallas kernel here")


# speedup vs baseline: 4.8531x; 4.8531x over previous
import functools

import jax
import jax.numpy as jnp
from jax import lax
from jax.experimental import pallas as pl
from jax.experimental.pallas import tpu as pltpu
from jax.experimental.pallas import tpu_sc as plsc

N_NODES = 10000
N_EDGES = 320000
D_FEAT = 128

NC = 2
NS = 16
NW = NC * NS
L = 16

CHUNK = 128
NCHUNK = -(-N_EDGES // (NW * CHUNK))
E_PAD = NW * NCHUNK * CHUNK
ROWS_PER_TILE = (N_NODES // NS) // 8 * 8
ROWS_REM = N_NODES - NS * ROWS_PER_TILE

_mesh = plsc.VectorSubcoreMesh(
    core_axis_name="c", subcore_axis_name="s", num_cores=NC, num_subcores=NS
)


@functools.partial(
    pl.kernel,
    out_type=jax.ShapeDtypeStruct((NC, N_NODES, D_FEAT), jnp.float32),
    mesh=_mesh,
    scratch_types=[
        pltpu.VMEM((NCHUNK, CHUNK), jnp.int32),
        pltpu.VMEM((NCHUNK, CHUNK), jnp.int32),
        pltpu.VMEM((NCHUNK * CHUNK,), jnp.float32),
        pltpu.VMEM((CHUNK, D_FEAT), jnp.float32),
        pltpu.VMEM_SHARED((N_NODES, D_FEAT), jnp.float32),
        pltpu.SemaphoreType.DMA,
    ],
)
def _sc_aggregate(cols_h, rows_h, w_h, feats_h, partial_h,
                  cols_v, rows_v, w_v, gbuf, acc, sem):
    cid = lax.axis_index("c")
    sid = lax.axis_index("s")
    wid = cid * NS + sid

    pltpu.sync_copy(cols_h.at[wid], cols_v)
    pltpu.sync_copy(rows_h.at[wid], rows_v)
    pltpu.sync_copy(w_h.at[wid], w_v)

    zero16 = jnp.zeros((L,), jnp.float32)

    def _zrow(i, carry):
        for r in range(D_FEAT // L):
            gbuf[i, pl.ds(r * L, L)] = zero16
        return carry

    lax.fori_loop(0, CHUNK, _zrow, 0)
    base = sid * ROWS_PER_TILE
    full, rem = divmod(ROWS_PER_TILE, CHUNK)
    for k in range(full):
        pltpu.sync_copy(gbuf, acc.at[pl.ds(base + k * CHUNK, CHUNK)])
    if rem:
        pltpu.sync_copy(gbuf.at[pl.ds(0, rem)],
                        acc.at[pl.ds(base + full * CHUNK, rem)])

    @pl.when(sid == NS - 1)
    def _zero_tail():
        pltpu.sync_copy(gbuf.at[pl.ds(0, ROWS_REM)],
                        acc.at[pl.ds(NS * ROWS_PER_TILE, ROWS_REM)])

    plsc.subcore_barrier()

    def _chunk(j, carry):
        pltpu.async_copy(feats_h.at[cols_v.at[j]], gbuf, sem).wait()

        jbase = j * CHUNK

        def _group(g, carry):
            w16 = w_v[pl.ds(jbase + g * L, L)]
            for i in range(L):
                e = g * L + i
                wi = jnp.broadcast_to(w16[i], (L,))
                for r in range(D_FEAT // L):
                    gbuf[e, pl.ds(r * L, L)] = gbuf[e, pl.ds(r * L, L)] * wi
            return carry

        lax.fori_loop(0, CHUNK // L, _group, 0)

        pltpu.sync_copy(gbuf, acc.at[rows_v.at[j]], add=True)
        return carry

    lax.fori_loop(0, NCHUNK, _chunk, 0)

    plsc.subcore_barrier()
    pltpu.sync_copy(acc.at[pl.ds(base, ROWS_PER_TILE)],
                    partial_h.at[cid, pl.ds(base, ROWS_PER_TILE)])

    @pl.when(sid == NS - 1)
    def _flush_tail():
        pltpu.sync_copy(
            acc.at[pl.ds(NS * ROWS_PER_TILE, ROWS_REM)],
            partial_h.at[cid, pl.ds(NS * ROWS_PER_TILE, ROWS_REM)])


def _combine_body(p_ref, o_ref):
    o_ref[...] = p_ref[0] + p_ref[1]


_ROWS_BLK = 1000


@jax.jit
def _combine(partial):
    return pl.pallas_call(
        _combine_body,
        out_shape=jax.ShapeDtypeStruct((N_NODES, D_FEAT), jnp.float32),
        grid=(N_NODES // _ROWS_BLK,),
        in_specs=[pl.BlockSpec((NC, _ROWS_BLK, D_FEAT), lambda i: (0, i, 0))],
        out_specs=pl.BlockSpec((_ROWS_BLK, D_FEAT), lambda i: (i, 0)),
    )(partial)


@jax.jit
def kernel(edge_index, edge_weight, feats):
    rows = edge_index[0].astype(jnp.int32)
    cols = edge_index[1].astype(jnp.int32)
    w = edge_weight.astype(jnp.float32)

    pad = E_PAD - N_EDGES
    rows = jnp.concatenate([rows, jnp.zeros((pad,), jnp.int32)])
    cols = jnp.concatenate([cols, jnp.zeros((pad,), jnp.int32)])
    w = jnp.concatenate([w, jnp.zeros((pad,), jnp.float32)])

    rows = rows.reshape(NW, NCHUNK, CHUNK)
    cols = cols.reshape(NW, NCHUNK, CHUNK)
    w = w.reshape(NW, NCHUNK * CHUNK)

    partial = _sc_aggregate(cols, rows, w, feats)
    return _combine(partial)
